# baseline (speedup 1.0000x reference)
import functools

import jax
import jax.numpy as jnp
import numpy as np
from jax import lax
from jax.experimental import pallas as pl
from jax.experimental.pallas import tpu as pltpu
from jax.experimental.pallas import tpu_sc as plsc

L = 16
T = 524288
F = 2
BASE_RES = 16
PER_LEVEL_SCALE = 1.3819128800392342
N = 262144
P1 = np.uint32(2654435761)
P2 = np.uint32(805459861)
RES = [int(np.floor(BASE_RES * PER_LEVEL_SCALE ** l)) for l in range(L)]

NC, NS, LANES = 2, 16, 16
NW = NC * NS
PTS_W = N // NW
C = 1024
NCHUNK = PTS_W // C
NGRP = C // LANES
R = 8 * C

CORNERS = [(dx, dy, dz) for dx in (0, 1) for dy in (0, 1) for dz in (0, 1)]


def _compute_level(l, xyz_v, idx_ref, w_ref):
    res = jnp.float32(RES[l])
    lofs = jnp.int32(l * T)
    mask = jnp.uint32(T - 1)

    def g_body(g, carry):
        s = g * LANES
        x = xyz_v[0, pl.ds(s, LANES)]
        y = xyz_v[1, pl.ds(s, LANES)]
        z = xyz_v[2, pl.ds(s, LANES)]
        xs = x * res
        ys = y * res
        zs = z * res
        xi = xs.astype(jnp.int32)
        yi = ys.astype(jnp.int32)
        zi = zs.astype(jnp.int32)
        wx = xs - xi.astype(jnp.float32)
        wy = ys - yi.astype(jnp.float32)
        wz = zs - zi.astype(jnp.float32)
        hx0 = xi.astype(jnp.uint32)
        hy0 = yi.astype(jnp.uint32) * P1
        hz0 = zi.astype(jnp.uint32) * P2
        hx1 = hx0 + jnp.uint32(1)
        hy1 = hy0 + P1
        hz1 = hz0 + P2
        ux = 1.0 - wx
        uy = 1.0 - wy
        uz = 1.0 - wz
        a00 = ux * uy
        a01 = ux * wy
        a10 = wx * uy
        a11 = wx * wy
        wxy = {(0, 0): a00, (0, 1): a01, (1, 0): a10, (1, 1): a11}
        lane = lax.iota(jnp.int32, LANES)
        feat = lane & 1
        dup_lo = (lane >> 1).reshape(LANES, 1)
        dup_hi = ((lane >> 1) + 8).reshape(LANES, 1)
        for c, (dx, dy, dz) in enumerate(CORNERS):
            hx = hx1 if dx else hx0
            hy = hy1 if dy else hy0
            hz = hz1 if dz else hz0
            h = (((hx ^ hy ^ hz) & mask)).astype(jnp.int32) + lofs
            e = h << 1
            elo, ehi = _dup8(e, dup_lo, dup_hi)
            idx_ref[pl.ds(2 * (c * C + s), LANES)] = elo + feat
            idx_ref[pl.ds(2 * (c * C + s) + LANES, LANES)] = ehi + feat
            w_ref[pl.ds(c * C + s, LANES)] = wxy[(dx, dy)] * (wz if dz else uz)
        return carry

    lax.fori_loop(0, NGRP, g_body, 0, unroll=False)


def _dup8(v, dup_lo, dup_hi):
    lo = lax.gather(v, dup_lo, _DUP_DNUMS, slice_sizes=(1,),
                    mode=lax.GatherScatterMode.PROMISE_IN_BOUNDS)
    hi = lax.gather(v, dup_hi, _DUP_DNUMS, slice_sizes=(1,),
                    mode=lax.GatherScatterMode.PROMISE_IN_BOUNDS)
    return lo, hi


_DUP_DNUMS = lax.GatherDimensionNumbers(
    offset_dims=(), collapsed_slice_dims=(0,), start_index_map=(0,))


def _accum_level(l, rows_ref, w_ref, enc_v):
    rf = rows_ref
    lane = lax.iota(jnp.int32, LANES)
    dup_lo = (lane >> 1).reshape(LANES, 1)
    dup_hi = ((lane >> 1) + 8).reshape(LANES, 1)

    def g_body(g, carry):
        s = g * LANES
        acc_a = None
        for c in range(8):
            r0 = c * C + s
            fa = rf[pl.ds(2 * r0, LANES)]
            fb = rf[pl.ds(2 * r0 + LANES, LANES)]
            wv = w_ref[pl.ds(r0, LANES)]
            wa, wb = _dup8(wv, dup_lo, dup_hi)
            if acc_a is None:
                acc_a = wa * fa
                acc_b = wb * fb
            else:
                acc_a = acc_a + wa * fa
                acc_b = acc_b + wb * fb
        enc_v[pl.ds(l * 2 * C + 2 * s, LANES)] = acc_a
        enc_v[pl.ds(l * 2 * C + 2 * s + LANES, LANES)] = acc_b
        return carry

    lax.fori_loop(0, NGRP, g_body, 0, unroll=False)


def _enc_body(xyz_hbm, tab_hbm, enc_hbm, xyz_v, idx_a, idx_b, rows_a, rows_b,
              w_a, w_b, enc_v, sem_a, sem_b):
    cid = lax.axis_index("c")
    sid = lax.axis_index("s")
    wid = sid * NC + cid
    base = wid * PTS_W

    idx_bufs = (idx_a, idx_b)
    rows_bufs = (rows_a, rows_b)
    w_bufs = (w_a, w_b)
    sems = (sem_a, sem_b)

    def chunk_body(ci, carry):
        off = base + ci * C
        pltpu.sync_copy(xyz_hbm.at[:, pl.ds(off, C)], xyz_v)
        _compute_level(0, xyz_v, idx_bufs[0], w_bufs[0])
        descs = [None, None]
        descs[0] = pltpu.async_copy(tab_hbm.at[idx_bufs[0]], rows_bufs[0], sems[0])
        for l in range(L):
            cur = l % 2
            nxt = (l + 1) % 2
            if l + 1 < L:
                _compute_level(l + 1, xyz_v, idx_bufs[nxt], w_bufs[nxt])
                descs[nxt] = pltpu.async_copy(
                    tab_hbm.at[idx_bufs[nxt]], rows_bufs[nxt], sems[nxt])
            descs[cur].wait()
            _accum_level(l, rows_bufs[cur], w_bufs[cur], enc_v)
        pltpu.sync_copy(enc_v, enc_hbm.at[pl.ds(off * (L * F), C * L * F)])
        return carry

    lax.fori_loop(0, NCHUNK, chunk_body, 0, unroll=False)


@functools.partial(
    pl.kernel,
    out_type=jax.ShapeDtypeStruct((N * L * F,), jnp.float32),
    mesh=plsc.VectorSubcoreMesh(core_axis_name="c", subcore_axis_name="s"),
    scratch_types=[
        pltpu.VMEM((3, C), jnp.float32),
        pltpu.VMEM((R * F,), jnp.int32),
        pltpu.VMEM((R * F,), jnp.int32),
        pltpu.VMEM((R * F,), jnp.float32),
        pltpu.VMEM((R * F,), jnp.float32),
        pltpu.VMEM((R,), jnp.float32),
        pltpu.VMEM((R,), jnp.float32),
        pltpu.VMEM((C * L * F,), jnp.float32),
        pltpu.SemaphoreType.DMA,
        pltpu.SemaphoreType.DMA,
    ],
)
def _encode_sc(xyz_hbm, tab_hbm, enc_hbm, *rest):
    _enc_body(xyz_hbm, tab_hbm, enc_hbm, *rest)


BLK = 8192


def _mlp_body(enc_ref, w0_ref, w1_ref, w2_ref, out_ref):
    h = jnp.dot(enc_ref[...], w0_ref[...], preferred_element_type=jnp.float32)
    h = jnp.maximum(h, 0.0)
    h = jnp.dot(h, w1_ref[...], preferred_element_type=jnp.float32)
    h = jnp.maximum(h, 0.0)
    out_ref[...] = jnp.dot(h, w2_ref[...], preferred_element_type=jnp.float32)


def _mlp(enc, W0, W1, W2):
    return pl.pallas_call(
        _mlp_body,
        grid=(N // BLK,),
        in_specs=[
            pl.BlockSpec((BLK, L * F), lambda i: (i, 0)),
            pl.BlockSpec((L * F, 64), lambda i: (0, 0)),
            pl.BlockSpec((64, 64), lambda i: (0, 0)),
            pl.BlockSpec((64, 1), lambda i: (0, 0)),
        ],
        out_specs=pl.BlockSpec((BLK, 1), lambda i: (i, 0)),
        out_shape=jax.ShapeDtypeStruct((N, 1), jnp.float32),
    )(enc, W0, W1, W2)


def kernel(inputs, table, W0, W1, W2):
    xyzT = inputs.T
    tab1 = table.reshape(L * T * F)
    enc = _encode_sc(xyzT, tab1)
    enc = enc.reshape(N // C, L, C, F).transpose(0, 2, 1, 3).reshape(N, L * F)
    return _mlp(enc, W0, W1, W2)


# speedup vs baseline: 1.1980x; 1.1980x over previous
import functools

import jax
import jax.numpy as jnp
import numpy as np
from jax import lax
from jax.experimental import pallas as pl
from jax.experimental.pallas import tpu as pltpu
from jax.experimental.pallas import tpu_sc as plsc

L = 16
T = 524288
F = 2
BASE_RES = 16
PER_LEVEL_SCALE = 1.3819128800392342
N = 262144
P1 = np.uint32(2654435761)
P2 = np.uint32(805459861)
RES = [int(np.floor(BASE_RES * PER_LEVEL_SCALE ** l)) for l in range(L)]

NC, NS, LANES = 2, 16, 16
NW = NC * NS
PTS_W = N // NW
C = 1024
NCHUNK = PTS_W // C
NGRP = C // LANES
R = 8 * C

CORNERS = [(dx, dy, dz) for dx in (0, 1) for dy in (0, 1) for dz in (0, 1)]


def _compute_level(l, xyz_v, idx_ref, w_ref):
    res = jnp.float32(RES[l])
    lofs = jnp.int32(l * T)
    mask = jnp.uint32(T - 1)

    def g_body(g, carry):
        s = g * LANES
        x = xyz_v[0, pl.ds(s, LANES)]
        y = xyz_v[1, pl.ds(s, LANES)]
        z = xyz_v[2, pl.ds(s, LANES)]
        xs = x * res
        ys = y * res
        zs = z * res
        xi = xs.astype(jnp.int32)
        yi = ys.astype(jnp.int32)
        zi = zs.astype(jnp.int32)
        wx = xs - xi.astype(jnp.float32)
        wy = ys - yi.astype(jnp.float32)
        wz = zs - zi.astype(jnp.float32)
        hx0 = xi.astype(jnp.uint32)
        hy0 = yi.astype(jnp.uint32) * P1
        hz0 = zi.astype(jnp.uint32) * P2
        hx1 = hx0 + jnp.uint32(1)
        hy1 = hy0 + P1
        hz1 = hz0 + P2
        ux = 1.0 - wx
        uy = 1.0 - wy
        uz = 1.0 - wz
        a00 = ux * uy
        a01 = ux * wy
        a10 = wx * uy
        a11 = wx * wy
        wxy = {(0, 0): a00, (0, 1): a01, (1, 0): a10, (1, 1): a11}
        for c, (dx, dy, dz) in enumerate(CORNERS):
            hx = hx1 if dx else hx0
            hy = hy1 if dy else hy0
            hz = hz1 if dz else hz0
            h = (((hx ^ hy ^ hz) & mask)).astype(jnp.int32) + lofs
            e = h << 1
            idx_ref[pl.ds(c * C + s, LANES)] = e
            idx_ref[pl.ds(R + c * C + s, LANES)] = e + 1
            w_ref[pl.ds(c * C + s, LANES)] = wxy[(dx, dy)] * (wz if dz else uz)
        return carry

    lax.fori_loop(0, NGRP, g_body, 0, unroll=False)


def _accum_level(l, rows_ref, w_ref, enc_v):
    def g_body(g, carry):
        s = g * LANES
        acc_a = None
        for c in range(8):
            r0 = c * C + s
            fa = rows_ref[pl.ds(r0, LANES)]
            fb = rows_ref[pl.ds(R + r0, LANES)]
            wv = w_ref[pl.ds(r0, LANES)]
            if acc_a is None:
                acc_a = wv * fa
                acc_b = wv * fb
            else:
                acc_a = acc_a + wv * fa
                acc_b = acc_b + wv * fb
        enc_v[2 * l, pl.ds(s, LANES)] = acc_a
        enc_v[2 * l + 1, pl.ds(s, LANES)] = acc_b
        return carry

    lax.fori_loop(0, NGRP, g_body, 0, unroll=False)


def _enc_body(xyz_hbm, tab_hbm, enc_hbm, xyz_v, idx_a, idx_b, rows_a, rows_b,
              w_a, w_b, enc_v, sem_a, sem_b):
    cid = lax.axis_index("c")
    sid = lax.axis_index("s")
    wid = sid * NC + cid
    base = wid * PTS_W

    idx_bufs = (idx_a, idx_b)
    rows_bufs = (rows_a, rows_b)
    w_bufs = (w_a, w_b)
    sems = (sem_a, sem_b)

    def chunk_body(ci, carry):
        off = base + ci * C
        pltpu.sync_copy(xyz_hbm.at[:, pl.ds(off, C)], xyz_v)
        _compute_level(0, xyz_v, idx_bufs[0], w_bufs[0])
        descs = [None, None]
        descs[0] = pltpu.async_copy(tab_hbm.at[idx_bufs[0]], rows_bufs[0], sems[0])
        for l in range(L):
            cur = l % 2
            nxt = (l + 1) % 2
            if l + 1 < L:
                _compute_level(l + 1, xyz_v, idx_bufs[nxt], w_bufs[nxt])
                descs[nxt] = pltpu.async_copy(
                    tab_hbm.at[idx_bufs[nxt]], rows_bufs[nxt], sems[nxt])
            descs[cur].wait()
            _accum_level(l, rows_bufs[cur], w_bufs[cur], enc_v)
        pltpu.sync_copy(enc_v, enc_hbm.at[:, pl.ds(off, C)])
        return carry

    lax.fori_loop(0, NCHUNK, chunk_body, 0, unroll=False)


@functools.partial(
    pl.kernel,
    out_type=jax.ShapeDtypeStruct((L * F, N), jnp.float32),
    mesh=plsc.VectorSubcoreMesh(core_axis_name="c", subcore_axis_name="s"),
    scratch_types=[
        pltpu.VMEM((3, C), jnp.float32),
        pltpu.VMEM((R * F,), jnp.int32),
        pltpu.VMEM((R * F,), jnp.int32),
        pltpu.VMEM((R * F,), jnp.float32),
        pltpu.VMEM((R * F,), jnp.float32),
        pltpu.VMEM((R,), jnp.float32),
        pltpu.VMEM((R,), jnp.float32),
        pltpu.VMEM((L * F, C), jnp.float32),
        pltpu.SemaphoreType.DMA,
        pltpu.SemaphoreType.DMA,
    ],
)
def _encode_sc(xyz_hbm, tab_hbm, enc_hbm, *rest):
    _enc_body(xyz_hbm, tab_hbm, enc_hbm, *rest)


BLK = 8192


def _mlp_body(enc_ref, w0_ref, w1_ref, w2_ref, out_ref):
    h = lax.dot_general(enc_ref[...], w0_ref[...], (((0,), (0,)), ((), ())),
                        preferred_element_type=jnp.float32)
    h = jnp.maximum(h, 0.0)
    h = jnp.dot(h, w1_ref[...], preferred_element_type=jnp.float32)
    h = jnp.maximum(h, 0.0)
    out_ref[...] = jnp.dot(h, w2_ref[...], preferred_element_type=jnp.float32)


def _mlp(enc, W0, W1, W2):
    return pl.pallas_call(
        _mlp_body,
        grid=(N // BLK,),
        in_specs=[
            pl.BlockSpec((L * F, BLK), lambda i: (0, i)),
            pl.BlockSpec((L * F, 64), lambda i: (0, 0)),
            pl.BlockSpec((64, 64), lambda i: (0, 0)),
            pl.BlockSpec((64, 1), lambda i: (0, 0)),
        ],
        out_specs=pl.BlockSpec((BLK, 1), lambda i: (i, 0)),
        out_shape=jax.ShapeDtypeStruct((N, 1), jnp.float32),
    )(enc, W0, W1, W2)


def kernel(inputs, table, W0, W1, W2):
    xyzT = inputs.T
    tab1 = table.reshape(L * T * F)
    enc = _encode_sc(xyzT, tab1)
    return _mlp(enc, W0, W1, W2)


# speedup vs baseline: 5.7578x; 4.8063x over previous
import functools

import jax
import jax.numpy as jnp
import numpy as np
from jax import lax
from jax.experimental import pallas as pl
from jax.experimental.pallas import tpu as pltpu
from jax.experimental.pallas import tpu_sc as plsc

L = 16
T = 524288
F = 2
BASE_RES = 16
PER_LEVEL_SCALE = 1.3819128800392342
N = 262144
P1 = np.uint32(2654435761)
P2 = np.uint32(805459861)
RES = [int(np.floor(BASE_RES * PER_LEVEL_SCALE ** l)) for l in range(L)]

NC, NS, LANES = 2, 16, 16
NW = NC * NS
PTS_W = N // NW
C = 1024
NCHUNK = PTS_W // C
NGRP = C // LANES
R = 8 * C

CORNERS = [(dx, dy, dz) for dx in (0, 1) for dy in (0, 1) for dz in (0, 1)]


def _compute_level(l, xyz_v, idx_ref, w_ref):
    res = jnp.float32(RES[l])
    lofs = jnp.int32(l * T)
    mask = jnp.uint32(T - 1)

    def g_body(g, carry):
        s = g * LANES
        x = xyz_v[0][pl.ds(s, LANES)]
        y = xyz_v[1][pl.ds(s, LANES)]
        z = xyz_v[2][pl.ds(s, LANES)]
        xs = x * res
        ys = y * res
        zs = z * res
        xi = xs.astype(jnp.int32)
        yi = ys.astype(jnp.int32)
        zi = zs.astype(jnp.int32)
        wx = xs - xi.astype(jnp.float32)
        wy = ys - yi.astype(jnp.float32)
        wz = zs - zi.astype(jnp.float32)
        hx0 = xi.astype(jnp.uint32)
        hy0 = yi.astype(jnp.uint32) * P1
        hz0 = zi.astype(jnp.uint32) * P2
        hx1 = hx0 + jnp.uint32(1)
        hy1 = hy0 + P1
        hz1 = hz0 + P2
        ux = 1.0 - wx
        uy = 1.0 - wy
        uz = 1.0 - wz
        a00 = ux * uy
        a01 = ux * wy
        a10 = wx * uy
        a11 = wx * wy
        wxy = {(0, 0): a00, (0, 1): a01, (1, 0): a10, (1, 1): a11}
        for c, (dx, dy, dz) in enumerate(CORNERS):
            hx = hx1 if dx else hx0
            hy = hy1 if dy else hy0
            hz = hz1 if dz else hz0
            h = (((hx ^ hy ^ hz) & mask)).astype(jnp.int32) + lofs
            idx_ref[pl.ds(c * C + s, LANES)] = h
            w_ref[pl.ds(c * C + s, LANES)] = wxy[(dx, dy)] * (wz if dz else uz)
        return carry

    lax.fori_loop(0, NGRP, g_body, 0, unroll=False)


def _accum_level(l, rows0_ref, rows1_ref, w_ref, enc_v):
    def g_body(g, carry):
        s = g * LANES
        acc_a = None
        for c in range(8):
            r0 = c * C + s
            fa = rows0_ref[pl.ds(r0, LANES)]
            fb = rows1_ref[pl.ds(r0, LANES)]
            wv = w_ref[pl.ds(r0, LANES)]
            if acc_a is None:
                acc_a = wv * fa
                acc_b = wv * fb
            else:
                acc_a = acc_a + wv * fa
                acc_b = acc_b + wv * fb
        enc_v[2 * l, pl.ds(s, LANES)] = acc_a
        enc_v[2 * l + 1, pl.ds(s, LANES)] = acc_b
        return carry

    lax.fori_loop(0, NGRP, g_body, 0, unroll=False)


def _enc_body(x_hbm, y_hbm, z_hbm, tab0_hbm, tab1_hbm, enc_hbm,
              xv, yv, zv, idx_a, idx_b,
              rows_a0, rows_a1, rows_b0, rows_b1, w_a, w_b, enc_v,
              sem_a, sem_b):
    cid = lax.axis_index("c")
    sid = lax.axis_index("s")
    wid = sid * NC + cid
    base = wid * PTS_W

    xyz_v = (xv, yv, zv)
    idx_bufs = (idx_a, idx_b)
    rows_bufs = ((rows_a0, rows_a1), (rows_b0, rows_b1))
    w_bufs = (w_a, w_b)
    sems = (sem_a, sem_b)

    def fire(slot):
        d0 = pltpu.async_copy(tab0_hbm.at[idx_bufs[slot]],
                              rows_bufs[slot][0], sems[slot])
        d1 = pltpu.async_copy(tab1_hbm.at[idx_bufs[slot]],
                              rows_bufs[slot][1], sems[slot])
        return (d0, d1)

    def chunk_body(ci, carry):
        off = base + ci * C
        pltpu.sync_copy(x_hbm.at[pl.ds(off, C)], xv)
        pltpu.sync_copy(y_hbm.at[pl.ds(off, C)], yv)
        pltpu.sync_copy(z_hbm.at[pl.ds(off, C)], zv)
        _compute_level(0, xyz_v, idx_bufs[0], w_bufs[0])
        descs = [None, None]
        descs[0] = fire(0)
        for l in range(L):
            cur = l % 2
            nxt = (l + 1) % 2
            if l + 1 < L:
                _compute_level(l + 1, xyz_v, idx_bufs[nxt], w_bufs[nxt])
                descs[nxt] = fire(nxt)
            descs[cur][0].wait()
            descs[cur][1].wait()
            _accum_level(l, rows_bufs[cur][0], rows_bufs[cur][1],
                         w_bufs[cur], enc_v)
        pltpu.sync_copy(enc_v, enc_hbm.at[:, pl.ds(off, C)])
        return carry

    lax.fori_loop(0, NCHUNK, chunk_body, 0, unroll=False)


@functools.partial(
    pl.kernel,
    out_type=jax.ShapeDtypeStruct((L * F, N), jnp.float32),
    mesh=plsc.VectorSubcoreMesh(core_axis_name="c", subcore_axis_name="s"),
    scratch_types=[
        pltpu.VMEM((C,), jnp.float32),
        pltpu.VMEM((C,), jnp.float32),
        pltpu.VMEM((C,), jnp.float32),
        pltpu.VMEM((R,), jnp.int32),
        pltpu.VMEM((R,), jnp.int32),
        pltpu.VMEM((R,), jnp.float32),
        pltpu.VMEM((R,), jnp.float32),
        pltpu.VMEM((R,), jnp.float32),
        pltpu.VMEM((R,), jnp.float32),
        pltpu.VMEM((R,), jnp.float32),
        pltpu.VMEM((R,), jnp.float32),
        pltpu.VMEM((L * F, C), jnp.float32),
        pltpu.SemaphoreType.DMA,
        pltpu.SemaphoreType.DMA,
    ],
)
def _encode_sc(x_hbm, y_hbm, z_hbm, tab0_hbm, tab1_hbm, enc_hbm, *rest):
    _enc_body(x_hbm, y_hbm, z_hbm, tab0_hbm, tab1_hbm, enc_hbm, *rest)


BLK = 8192


def _mlp_body(enc_ref, w0_ref, w1_ref, w2_ref, out_ref):
    h = lax.dot_general(enc_ref[...], w0_ref[...], (((0,), (0,)), ((), ())),
                        preferred_element_type=jnp.float32)
    h = jnp.maximum(h, 0.0)
    h = jnp.dot(h, w1_ref[...], preferred_element_type=jnp.float32)
    h = jnp.maximum(h, 0.0)
    out_ref[...] = jnp.dot(h, w2_ref[...], preferred_element_type=jnp.float32)


def _mlp(enc, W0, W1, W2):
    return pl.pallas_call(
        _mlp_body,
        grid=(N // BLK,),
        in_specs=[
            pl.BlockSpec((L * F, BLK), lambda i: (0, i)),
            pl.BlockSpec((L * F, 64), lambda i: (0, 0)),
            pl.BlockSpec((64, 64), lambda i: (0, 0)),
            pl.BlockSpec((64, 1), lambda i: (0, 0)),
        ],
        out_specs=pl.BlockSpec((BLK, 1), lambda i: (i, 0)),
        out_shape=jax.ShapeDtypeStruct((N, 1), jnp.float32),
    )(enc, W0, W1, W2)


def kernel(inputs, table, W0, W1, W2):
    x1 = inputs[:, 0]
    y1 = inputs[:, 1]
    z1 = inputs[:, 2]
    tab0 = table[:, :, 0].reshape(L * T)
    tab1 = table[:, :, 1].reshape(L * T)
    enc = _encode_sc(x1, y1, z1, tab0, tab1)
    return _mlp(enc, W0, W1, W2)
